# baseline (speedup 1.0000x reference)
import jax
import jax.numpy as jnp
from jax import lax
from jax.experimental import pallas as pl
from jax.experimental.pallas import tpu as pltpu
from jax.experimental.pallas import tpu_sc as plsc

N, D_IN, D_HID, N_CLUSTERS, D_AUX = 1024, 1024, 512, 1024, 64
C_SC = 0
C_TC = N_CLUSTERS - C_SC
BM = 128
BNE, BCE = 16, 256
BS = 128


def _mlp_body(x_ref, w1_ref, b1_ref, w2_ref, b2_ref, z_ref):
    h = jnp.maximum(
        jnp.dot(x_ref[...], w1_ref[...], preferred_element_type=jnp.float32)
        + b1_ref[...], 0.0)
    z_ref[...] = jnp.dot(h, w2_ref[...],
                         preferred_element_type=jnp.float32) + b2_ref[...]


def _einsum_body(z_ref, cc_ref, out_ref):
    z = z_ref[...]
    out_ref[...] = jnp.sum(cc_ref[...] * z[:, None, :], axis=2)


def _lsm_body(*refs):
    ins, out_ref = refs[:-1], refs[-1]
    l = jnp.concatenate([r[...] for r in ins], axis=1)
    lt = l.T
    m = jnp.max(lt, axis=0)
    e = jnp.exp(lt - m[None, :])
    s = jnp.sum(e, axis=0)
    r = lt - (m + jnp.log(s))[None, :]
    out_ref[...] = r.T


def _mlp(inputs, W1, b1, W2, b2):
    return pl.pallas_call(
        _mlp_body,
        grid=(N // BM,),
        in_specs=[
            pl.BlockSpec((BM, D_IN), lambda i: (i, 0)),
            pl.BlockSpec((D_IN, D_HID), lambda i: (0, 0)),
            pl.BlockSpec((1, D_HID), lambda i: (0, 0)),
            pl.BlockSpec((D_HID, D_AUX), lambda i: (0, 0)),
            pl.BlockSpec((1, D_AUX), lambda i: (0, 0)),
        ],
        out_specs=pl.BlockSpec((BM, D_AUX), lambda i: (i, 0)),
        out_shape=jax.ShapeDtypeStruct((N, D_AUX), jnp.float32),
    )(inputs, W1, b1.reshape(1, D_HID), W2, b2.reshape(1, D_AUX))


def _einsum_tc(z, cluster_centers):
    c0 = C_SC // BCE
    return pl.pallas_call(
        _einsum_body,
        grid=(N // BNE, C_TC // BCE),
        in_specs=[
            pl.BlockSpec((BNE, D_AUX), lambda i, j: (i, 0)),
            pl.BlockSpec((BNE, BCE, D_AUX), lambda i, j: (i, c0 + j, 0)),
        ],
        out_specs=pl.BlockSpec((BNE, BCE), lambda i, j: (i, j)),
        out_shape=jax.ShapeDtypeStruct((N, C_TC), jnp.float32),
    )(z, cluster_centers)


def _log_softmax(parts):
    n_in = len(parts)
    widths = [p.shape[1] for p in parts]
    return pl.pallas_call(
        _lsm_body,
        grid=(N // BS,),
        in_specs=[pl.BlockSpec((BS, w), lambda i: (i, 0)) for w in widths],
        out_specs=pl.BlockSpec((BS, N_CLUSTERS), lambda i: (i, 0)),
        out_shape=jax.ShapeDtypeStruct((N, N_CLUSTERS), jnp.float32),
    )(*parts)


def kernel(inputs, cluster_centers, W1, b1, W2, b2):
    z = _mlp(inputs, W1, b1, W2, b2)
    return _einsum_tc(z, cluster_centers)
    parts = []
    if C_SC > 0:
        parts.append(_einsum_sc(z, cluster_centers))
    if C_TC > 0:
        parts.append(_einsum_tc(z, cluster_centers))
    return _log_softmax(parts)


# speedup vs baseline: 1.1339x; 1.1339x over previous
import jax
import jax.numpy as jnp
from jax import lax
from jax.experimental import pallas as pl
from jax.experimental.pallas import tpu as pltpu
from jax.experimental.pallas import tpu_sc as plsc

N, D_IN, D_HID, N_CLUSTERS, D_AUX = 1024, 1024, 512, 1024, 64
C_SC = 0
C_TC = N_CLUSTERS - C_SC
BM = 128
BNE = 64
BS = 128


def _mlp_body(x_ref, w1_ref, b1_ref, w2_ref, b2_ref, z_ref):
    h = jnp.maximum(
        jnp.dot(x_ref[...], w1_ref[...], preferred_element_type=jnp.float32)
        + b1_ref[...], 0.0)
    z_ref[...] = jnp.dot(h, w2_ref[...],
                         preferred_element_type=jnp.float32) + b2_ref[...]


KCH = 8192
CPK = KCH // D_AUX


def _einsum_body(z_ref, cc_ref, s_ref, out_ref):
    zz = z_ref[...]
    zt = pltpu.repeat(zz, KCH // 128, axis=1)
    prod = (cc_ref[...] * zt).astype(jnp.bfloat16)
    out_ref[...] = jnp.dot(prod, s_ref[...],
                           preferred_element_type=jnp.float32)


def _lsm_body(*refs):
    ins, out_ref = refs[:-1], refs[-1]
    l = jnp.concatenate([r[...] for r in ins], axis=1)
    lt = l.T
    m = jnp.max(lt, axis=0)
    e = jnp.exp(lt - m[None, :])
    s = jnp.sum(e, axis=0)
    r = lt - (m + jnp.log(s))[None, :]
    out_ref[...] = r.T


def _mlp(inputs, W1, b1, W2, b2):
    return pl.pallas_call(
        _mlp_body,
        grid=(N // BM,),
        in_specs=[
            pl.BlockSpec((BM, D_IN), lambda i: (i, 0)),
            pl.BlockSpec((D_IN, D_HID), lambda i: (0, 0)),
            pl.BlockSpec((1, D_HID), lambda i: (0, 0)),
            pl.BlockSpec((D_HID, D_AUX), lambda i: (0, 0)),
            pl.BlockSpec((1, D_AUX), lambda i: (0, 0)),
        ],
        out_specs=pl.BlockSpec((BM, D_AUX), lambda i: (i, 0)),
        out_shape=jax.ShapeDtypeStruct((N, D_AUX), jnp.float32),
    )(inputs, W1, b1.reshape(1, D_HID), W2, b2.reshape(1, D_AUX))


def _einsum_tc(z, cluster_centers):
    cc2 = cluster_centers.reshape(N, N_CLUSTERS * D_AUX)
    zcat = jnp.concatenate([z, z], axis=1)
    sel = (jnp.arange(KCH)[:, None] // D_AUX
           == jnp.arange(CPK)[None, :]).astype(jnp.bfloat16)
    j0 = C_SC * D_AUX // KCH
    return pl.pallas_call(
        _einsum_body,
        grid=(N // BNE, C_TC * D_AUX // KCH),
        in_specs=[
            pl.BlockSpec((BNE, 128), lambda i, j: (i, 0)),
            pl.BlockSpec((BNE, KCH), lambda i, j: (i, j0 + j)),
            pl.BlockSpec((KCH, CPK), lambda i, j: (0, 0)),
        ],
        out_specs=pl.BlockSpec((BNE, CPK), lambda i, j: (i, j)),
        out_shape=jax.ShapeDtypeStruct((N, C_TC), jnp.float32),
    )(zcat, cc2, sel)


def _log_softmax(parts):
    n_in = len(parts)
    widths = [p.shape[1] for p in parts]
    return pl.pallas_call(
        _lsm_body,
        grid=(N // BS,),
        in_specs=[pl.BlockSpec((BS, w), lambda i: (i, 0)) for w in widths],
        out_specs=pl.BlockSpec((BS, N_CLUSTERS), lambda i: (i, 0)),
        out_shape=jax.ShapeDtypeStruct((N, N_CLUSTERS), jnp.float32),
    )(*parts)


def kernel(inputs, cluster_centers, W1, b1, W2, b2):
    z = _mlp(inputs, W1, b1, W2, b2)
    parts = []
    if C_SC > 0:
        parts.append(_einsum_sc(z, cluster_centers))
    if C_TC > 0:
        parts.append(_einsum_tc(z, cluster_centers))
    return _log_softmax(parts)


# speedup vs baseline: 6.6035x; 5.8237x over previous
import jax
import jax.numpy as jnp
from jax import lax
from jax.experimental import pallas as pl
from jax.experimental.pallas import tpu as pltpu
from jax.experimental.pallas import tpu_sc as plsc

N, D_IN, D_HID, N_CLUSTERS, D_AUX = 1024, 1024, 512, 1024, 64
C_SC = 0
C_TC = N_CLUSTERS - C_SC
BM = 128
BNE = 32
BS = 128


def _mlp_body(x_ref, w1_ref, b1_ref, w2_ref, b2_ref, z_ref):
    h = jnp.maximum(
        jnp.dot(x_ref[...], w1_ref[...], preferred_element_type=jnp.float32)
        + b1_ref[...], 0.0)
    z_ref[...] = jnp.dot(h, w2_ref[...],
                         preferred_element_type=jnp.float32) + b2_ref[...]


def _einsum_body(z_ref, cc_ref, out_ref):
    zt = z_ref[...][:, :, None]
    out_ref[...] = jnp.sum(cc_ref[...] * zt, axis=1)


def _lsm_body(*refs):
    ins, out_ref = refs[:-1], refs[-1]
    l = jnp.concatenate([r[...] for r in ins], axis=1)
    lt = l.T
    m = jnp.max(lt, axis=0)
    e = jnp.exp(lt - m[None, :])
    s = jnp.sum(e, axis=0)
    r = lt - (m + jnp.log(s))[None, :]
    out_ref[...] = r.T


def _mlp(inputs, W1, b1, W2, b2):
    return pl.pallas_call(
        _mlp_body,
        grid=(N // BM,),
        in_specs=[
            pl.BlockSpec((BM, D_IN), lambda i: (i, 0)),
            pl.BlockSpec((D_IN, D_HID), lambda i: (0, 0)),
            pl.BlockSpec((1, D_HID), lambda i: (0, 0)),
            pl.BlockSpec((D_HID, D_AUX), lambda i: (0, 0)),
            pl.BlockSpec((1, D_AUX), lambda i: (0, 0)),
        ],
        out_specs=pl.BlockSpec((BM, D_AUX), lambda i: (i, 0)),
        out_shape=jax.ShapeDtypeStruct((N, D_AUX), jnp.float32),
    )(inputs, W1, b1.reshape(1, D_HID), W2, b2.reshape(1, D_AUX))


BCE = 1024


def _einsum_tc(z, cc_t):
    j0 = C_SC // BCE
    return pl.pallas_call(
        _einsum_body,
        grid=(N // BNE, C_TC // BCE),
        in_specs=[
            pl.BlockSpec((BNE, D_AUX), lambda i, j: (i, 0)),
            pl.BlockSpec((BNE, D_AUX, BCE), lambda i, j: (i, 0, j0 + j)),
        ],
        out_specs=pl.BlockSpec((BNE, BCE), lambda i, j: (i, j)),
        out_shape=jax.ShapeDtypeStruct((N, C_TC), jnp.float32),
    )(z, cc_t)


def _log_softmax(parts):
    n_in = len(parts)
    widths = [p.shape[1] for p in parts]
    return pl.pallas_call(
        _lsm_body,
        grid=(N // BS,),
        in_specs=[pl.BlockSpec((BS, w), lambda i: (i, 0)) for w in widths],
        out_specs=pl.BlockSpec((BS, N_CLUSTERS), lambda i: (i, 0)),
        out_shape=jax.ShapeDtypeStruct((N, N_CLUSTERS), jnp.float32),
    )(*parts)


def kernel(inputs, cluster_centers, W1, b1, W2, b2):
    cc_t = jnp.swapaxes(cluster_centers, 1, 2)
    z = _mlp(inputs, W1, b1, W2, b2)
    parts = []
    if C_SC > 0:
        parts.append(_einsum_sc(z, cc_t))
    if C_TC > 0:
        parts.append(_einsum_tc(z, cc_t))
    return _log_softmax(parts)
